# baseline (speedup 1.0000x reference)
import functools

import jax
import jax.numpy as jnp
from jax import lax
from jax.experimental import pallas as pl
from jax.experimental.pallas import tpu as pltpu
from jax.experimental.pallas import tpu_sc as plsc

NC = 2
NS = 16
NW = NC * NS


def _make_segsum(n: int, e: int, chunk: int, q: int):
    assert e % NW == 0
    epw = e // NW
    assert epw % chunk == 0 and chunk % 8 == 0 and q % 8 == 0 and q < chunk
    nchunks = epw // chunk

    mesh = plsc.VectorSubcoreMesh(core_axis_name="c", subcore_axis_name="s")

    @functools.partial(
        pl.kernel,
        mesh=mesh,
        out_type=jax.ShapeDtypeStruct((NC, n), jnp.float32),
        scratch_types=[
            pltpu.VMEM((chunk,), jnp.int32),
            pltpu.VMEM((chunk,), jnp.int32),
            pltpu.VMEM((chunk,), jnp.int32),
            pltpu.VMEM((chunk,), jnp.float32),
            pltpu.VMEM((chunk,), jnp.float32),
            pltpu.VMEM_SHARED((n,), jnp.float32),
            pltpu.VMEM_SHARED((n,), jnp.float32),
            pltpu.SemaphoreType.DMA,
            pltpu.SemaphoreType.DMA,
            pltpu.SemaphoreType.DMA,
        ],
    )
    def segsum(pp_hbm, zeros_hbm, edge_hbm, out_hbm,
               idx_s, idx_d0, idx_d1, vals0, vals1,
               pp_sh, acc_sh, sem_ld, sem_g, sem_sc):
        c = lax.axis_index("c")
        s = lax.axis_index("s")
        idx_d = (idx_d0, idx_d1)
        vals = (vals0, vals1)

        base = (c * NS + s) * epw

        def start_loads(i, b):
            off = base + i * chunk
            return (
                pltpu.async_copy(edge_hbm.at[pl.ds(off, chunk)], idx_s,
                                 sem_ld),
                pltpu.async_copy(edge_hbm.at[pl.ds(e + off, chunk)], idx_d[b],
                                 sem_ld),
            )

        loads = start_loads(0, 0)

        @pl.when(s == 0)
        def _stage():
            pltpu.sync_copy(pp_hbm, pp_sh)
            pltpu.sync_copy(zeros_hbm, acc_sh)

        plsc.subcore_barrier()

        scatter = None
        for i in range(nchunks):
            b = i % 2
            nb = (i + 1) % 2
            for ld in loads:
                ld.wait()
            gh = pltpu.async_copy(pp_hbm.at[idx_s.at[pl.ds(0, q)]],
                                  vals[b].at[pl.ds(0, q)], sem_g)
            pltpu.sync_copy(pp_sh.at[idx_s.at[pl.ds(q, chunk - q)]],
                            vals[b].at[pl.ds(q, chunk - q)])
            gh.wait()
            if scatter is not None:
                scatter.wait()
            if i + 1 < nchunks:
                loads = start_loads(i + 1, nb)
            scatter = pltpu.async_copy(vals[b], acc_sh.at[idx_d[b]],
                                       sem_sc, add=True)
        scatter.wait()

        plsc.subcore_barrier()

        @pl.when(s == 0)
        def _writeback():
            pltpu.sync_copy(acc_sh, out_hbm.at[c])

    return segsum


def _apply_body(p_ref, s_ref, o_ref):
    x = p_ref[0] + p_ref[1] - 1.0
    o_ref[...] = x * x * s_ref[...]


def kernel(pred_prob, special_cost, edge_index):
    n = pred_prob.shape[0]
    e = edge_index.shape[1]
    pp = pred_prob.reshape(n)
    if edge_index.dtype != jnp.int32:
        edge_index = edge_index.astype(jnp.int32)
    zeros = jnp.zeros((n,), jnp.float32)

    segsum = _make_segsum(n, e, chunk=20000, q=6664)
    partial = segsum(pp, zeros, edge_index.reshape(2 * e))

    p = (-n) % (8 * 128)
    np_ = n + p
    rows = np_ // 128
    p3 = jnp.pad(partial, ((0, 0), (0, p))).reshape(NC, rows, 128)
    s2 = jnp.pad(special_cost, (0, p)).reshape(rows, 128)
    out = pl.pallas_call(
        _apply_body,
        out_shape=jax.ShapeDtypeStruct((rows, 128), jnp.float32),
    )(p3, s2)
    return out.reshape(np_)[:n]


# speedup vs baseline: 1.0325x; 1.0325x over previous
import functools

import jax
import jax.numpy as jnp
from jax import lax
from jax.experimental import pallas as pl
from jax.experimental.pallas import tpu as pltpu
from jax.experimental.pallas import tpu_sc as plsc

NC = 2
NS = 16
NW = NC * NS


def _make_segsum(n: int, e: int, chunk: int, q: int):
    assert e % NW == 0
    epw = e // NW
    assert epw % chunk == 0 and chunk % 8 == 0 and q % 8 == 0 and q < chunk
    nchunks = epw // chunk

    mesh = plsc.VectorSubcoreMesh(core_axis_name="c", subcore_axis_name="s")

    @functools.partial(
        pl.kernel,
        mesh=mesh,
        out_type=jax.ShapeDtypeStruct((NC, n), jnp.float32),
        scratch_types=[
            pltpu.VMEM((chunk,), jnp.int32),
            pltpu.VMEM((chunk,), jnp.int32),
            pltpu.VMEM((chunk,), jnp.int32),
            pltpu.VMEM((chunk,), jnp.float32),
            pltpu.VMEM((chunk,), jnp.float32),
            pltpu.VMEM_SHARED((n,), jnp.float32),
            pltpu.VMEM_SHARED((n,), jnp.float32),
            pltpu.SemaphoreType.DMA,
            pltpu.SemaphoreType.DMA,
            pltpu.SemaphoreType.DMA,
        ],
    )
    def segsum(pp_hbm, zeros_hbm, edge_hbm, out_hbm,
               idx_s, idx_d0, idx_d1, vals0, vals1,
               pp_sh, acc_sh, sem_ld, sem_g, sem_sc):
        c = lax.axis_index("c")
        s = lax.axis_index("s")
        idx_d = (idx_d0, idx_d1)
        vals = (vals0, vals1)

        base = (c * NS + s) * epw

        def start_loads(i, b):
            off = base + i * chunk
            return (
                pltpu.async_copy(edge_hbm.at[pl.ds(off, chunk)], idx_s,
                                 sem_ld),
                pltpu.async_copy(edge_hbm.at[pl.ds(e + off, chunk)], idx_d[b],
                                 sem_ld),
            )

        loads = start_loads(0, 0)

        @pl.when(s == 0)
        def _stage():
            pltpu.sync_copy(pp_hbm, pp_sh)
            pltpu.sync_copy(zeros_hbm, acc_sh)

        plsc.subcore_barrier()

        scatter = None
        for i in range(nchunks):
            b = i % 2
            nb = (i + 1) % 2
            for ld in loads:
                ld.wait()
            gh = pltpu.async_copy(pp_hbm.at[idx_s.at[pl.ds(0, q)]],
                                  vals[b].at[pl.ds(0, q)], sem_g)
            pltpu.sync_copy(pp_sh.at[idx_s.at[pl.ds(q, chunk - q)]],
                            vals[b].at[pl.ds(q, chunk - q)])
            gh.wait()
            if scatter is not None:
                scatter.wait()
            if i + 1 < nchunks:
                loads = start_loads(i + 1, nb)
            scatter = pltpu.async_copy(vals[b], acc_sh.at[idx_d[b]],
                                       sem_sc, add=True)
        scatter.wait()

        plsc.subcore_barrier()

        @pl.when(s == 0)
        def _writeback():
            pltpu.sync_copy(acc_sh, out_hbm.at[c])

    return segsum


def _apply_body(p_ref, s_ref, o_ref):
    x = p_ref[0] + p_ref[1] - 1.0
    o_ref[...] = x * x * s_ref[...]


def kernel(pred_prob, special_cost, edge_index):
    n = pred_prob.shape[0]
    e = edge_index.shape[1]
    pp = pred_prob.reshape(n)
    if edge_index.dtype != jnp.int32:
        edge_index = edge_index.astype(jnp.int32)
    zeros = jnp.zeros((n,), jnp.float32)

    segsum = _make_segsum(n, e, chunk=20000, q=4000)
    partial = segsum(pp, zeros, edge_index.reshape(2 * e))

    p = (-n) % (8 * 128)
    np_ = n + p
    rows = np_ // 128
    p3 = jnp.pad(partial, ((0, 0), (0, p))).reshape(NC, rows, 128)
    s2 = jnp.pad(special_cost, (0, p)).reshape(rows, 128)
    out = pl.pallas_call(
        _apply_body,
        out_shape=jax.ShapeDtypeStruct((rows, 128), jnp.float32),
    )(p3, s2)
    return out.reshape(np_)[:n]


# speedup vs baseline: 1.0515x; 1.0184x over previous
import functools

import jax
import jax.numpy as jnp
from jax import lax
from jax.experimental import pallas as pl
from jax.experimental.pallas import tpu as pltpu
from jax.experimental.pallas import tpu_sc as plsc

NC = 2
NS = 16
NW = NC * NS


def _make_segsum(n: int, e: int, chunk: int, q: int):
    assert e % NW == 0
    epw = e // NW
    assert epw % chunk == 0 and chunk % 8 == 0 and q % 8 == 0 and q < chunk
    nchunks = epw // chunk

    mesh = plsc.VectorSubcoreMesh(core_axis_name="c", subcore_axis_name="s")

    @functools.partial(
        pl.kernel,
        mesh=mesh,
        out_type=jax.ShapeDtypeStruct((NC, n), jnp.float32),
        scratch_types=[
            pltpu.VMEM((chunk,), jnp.int32),
            pltpu.VMEM((chunk,), jnp.int32),
            pltpu.VMEM((chunk,), jnp.int32),
            pltpu.VMEM((chunk,), jnp.float32),
            pltpu.VMEM((chunk,), jnp.float32),
            pltpu.VMEM_SHARED((n,), jnp.float32),
            pltpu.VMEM_SHARED((n,), jnp.float32),
            pltpu.SemaphoreType.DMA,
            pltpu.SemaphoreType.DMA,
            pltpu.SemaphoreType.DMA,
        ],
    )
    def segsum(pp_hbm, zeros_hbm, edge_hbm, out_hbm,
               idx_s, idx_d0, idx_d1, vals0, vals1,
               pp_sh, acc_sh, sem_ld, sem_g, sem_sc):
        c = lax.axis_index("c")
        s = lax.axis_index("s")
        idx_d = (idx_d0, idx_d1)
        vals = (vals0, vals1)

        base = (c * NS + s) * epw

        def start_loads(i, b):
            off = base + i * chunk
            return (
                pltpu.async_copy(edge_hbm.at[pl.ds(off, chunk)], idx_s,
                                 sem_ld),
                pltpu.async_copy(edge_hbm.at[pl.ds(e + off, chunk)], idx_d[b],
                                 sem_ld),
            )

        loads = start_loads(0, 0)

        @pl.when(s == 0)
        def _stage():
            pltpu.sync_copy(pp_hbm, pp_sh)
            pltpu.sync_copy(zeros_hbm, acc_sh)

        plsc.subcore_barrier()

        scatter = None
        for i in range(nchunks):
            b = i % 2
            nb = (i + 1) % 2
            for ld in loads:
                ld.wait()
            gh = pltpu.async_copy(pp_hbm.at[idx_s.at[pl.ds(0, q)]],
                                  vals[b].at[pl.ds(0, q)], sem_g)
            pltpu.sync_copy(pp_sh.at[idx_s.at[pl.ds(q, chunk - q)]],
                            vals[b].at[pl.ds(q, chunk - q)])
            gh.wait()
            if scatter is not None:
                scatter.wait()
            if i + 1 < nchunks:
                loads = start_loads(i + 1, nb)
            scatter = pltpu.async_copy(vals[b], acc_sh.at[idx_d[b]],
                                       sem_sc, add=True)
        scatter.wait()

        plsc.subcore_barrier()

        @pl.when(s == 0)
        def _writeback():
            pltpu.sync_copy(acc_sh, out_hbm.at[c])

    return segsum


def _apply_body(p_ref, s_ref, o_ref):
    x = p_ref[0] + p_ref[1] - 1.0
    o_ref[...] = x * x * s_ref[...]


def kernel(pred_prob, special_cost, edge_index):
    n = pred_prob.shape[0]
    e = edge_index.shape[1]
    pp = pred_prob.reshape(n)
    if edge_index.dtype != jnp.int32:
        edge_index = edge_index.astype(jnp.int32)
    zeros = jnp.zeros((n,), jnp.float32)

    segsum = _make_segsum(n, e, chunk=20000, q=5000)
    partial = segsum(pp, zeros, edge_index.reshape(2 * e))

    return pl.pallas_call(
        _apply_body,
        out_shape=jax.ShapeDtypeStruct((n,), jnp.float32),
    )(partial, special_cost)


# speedup vs baseline: 1.0565x; 1.0048x over previous
import functools

import jax
import jax.numpy as jnp
from jax import lax
from jax.experimental import pallas as pl
from jax.experimental.pallas import tpu as pltpu
from jax.experimental.pallas import tpu_sc as plsc

NC = 2
NS = 16
NW = NC * NS


def _make_segsum(n: int, e: int, chunk: int, q: int):
    assert e % NW == 0
    epw = e // NW
    assert epw % chunk == 0 and chunk % 8 == 0 and q % 8 == 0 and q < chunk
    nchunks = epw // chunk

    mesh = plsc.VectorSubcoreMesh(core_axis_name="c", subcore_axis_name="s")

    @functools.partial(
        pl.kernel,
        mesh=mesh,
        out_type=jax.ShapeDtypeStruct((NC, n), jnp.float32),
        scratch_types=[
            pltpu.VMEM((chunk,), jnp.int32),
            pltpu.VMEM((chunk,), jnp.int32),
            pltpu.VMEM((chunk,), jnp.int32),
            pltpu.VMEM((chunk,), jnp.float32),
            pltpu.VMEM((chunk,), jnp.float32),
            pltpu.VMEM_SHARED((n,), jnp.float32),
            pltpu.VMEM_SHARED((n,), jnp.float32),
            pltpu.SemaphoreType.DMA,
            pltpu.SemaphoreType.DMA,
            pltpu.SemaphoreType.DMA,
        ],
    )
    def segsum(pp_hbm, zeros_hbm, edge_hbm, out_hbm,
               idx_s, idx_d0, idx_d1, vals0, vals1,
               pp_sh, acc_sh, sem_ld, sem_g, sem_sc):
        c = lax.axis_index("c")
        s = lax.axis_index("s")
        idx_d = (idx_d0, idx_d1)
        vals = (vals0, vals1)

        base = (c * NS + s) * epw

        def start_loads(i, b):
            off = base + i * chunk
            return (
                pltpu.async_copy(edge_hbm.at[pl.ds(off, chunk)], idx_s,
                                 sem_ld),
                pltpu.async_copy(edge_hbm.at[pl.ds(e + off, chunk)], idx_d[b],
                                 sem_ld),
            )

        loads = start_loads(0, 0)

        @pl.when(s == 0)
        def _stage():
            pltpu.sync_copy(pp_hbm, pp_sh)
            pltpu.sync_copy(zeros_hbm, acc_sh)

        plsc.subcore_barrier()

        scatter = None
        for i in range(nchunks):
            b = i % 2
            nb = (i + 1) % 2
            for ld in loads:
                ld.wait()
            gh = pltpu.async_copy(pp_hbm.at[idx_s.at[pl.ds(0, q)]],
                                  vals[b].at[pl.ds(0, q)], sem_g)
            pltpu.sync_copy(pp_sh.at[idx_s.at[pl.ds(q, chunk - q)]],
                            vals[b].at[pl.ds(q, chunk - q)])
            gh.wait()
            if scatter is not None:
                scatter.wait()
            if i + 1 < nchunks:
                loads = start_loads(i + 1, nb)
            scatter = pltpu.async_copy(vals[b], acc_sh.at[idx_d[b]],
                                       sem_sc, add=True)
        scatter.wait()

        plsc.subcore_barrier()

        @pl.when(s == 0)
        def _writeback():
            pltpu.sync_copy(acc_sh, out_hbm.at[c])

    return segsum


def _apply_body(p_ref, s_ref, o_ref):
    x = p_ref[0] + p_ref[1] - 1.0
    o_ref[...] = x * x * s_ref[...]


def kernel(pred_prob, special_cost, edge_index):
    n = pred_prob.shape[0]
    e = edge_index.shape[1]
    pp = pred_prob.reshape(n)
    if edge_index.dtype != jnp.int32:
        edge_index = edge_index.astype(jnp.int32)
    zeros = jnp.zeros((n,), jnp.float32)

    segsum = _make_segsum(n, e, chunk=20000, q=5504)
    partial = segsum(pp, zeros, edge_index.reshape(2 * e))

    return pl.pallas_call(
        _apply_body,
        out_shape=jax.ShapeDtypeStruct((n,), jnp.float32),
    )(partial, special_cost)
